# baseline (speedup 1.0000x reference)
import functools

import jax
import jax.numpy as jnp
from jax import lax
from jax.experimental import pallas as pl
from jax.experimental.pallas import tpu as pltpu
from jax.experimental.pallas import tpu_sc as plsc

_NC = 2
_NS = 16
_NW = _NC * _NS
_L = 16
_CH = 128


def _make_transpose(v, d):
    mesh = plsc.VectorSubcoreMesh(core_axis_name="c", subcore_axis_name="s")
    n_win = v // 128
    n_tail = (v % 128) * d // 128
    rows_per_win = 128 * d // 128

    @functools.partial(
        pl.kernel,
        mesh=mesh,
        compiler_params=pltpu.CompilerParams(needs_layout_passes=False),
        out_type=jax.ShapeDtypeStruct(
            ((v * d + 127) // 128, 128), jnp.float32),
        scratch_types=[
            pltpu.VMEM((2, d, 128), jnp.float32),
            pltpu.VMEM((2, rows_per_win, 128), jnp.float32),
            pltpu.SemaphoreType.DMA,
            pltpu.SemaphoreType.DMA,
        ],
    )
    def transpose_kernel(t_hbm, tail_hbm, out_hbm, win_v, trs_v, gsem, ssem):
        wid = lax.axis_index("s") * _NC + lax.axis_index("c")
        lane = lax.iota(jnp.int32, _L)
        n_mine = (n_win - wid + _NW - 1) // _NW

        if n_tail:
            @pl.when(wid == 0)
            def _():
                stage = trs_v.at[0].at[pl.ds(0, n_tail)]
                pltpu.sync_copy(tail_hbm, stage)
                pltpu.sync_copy(
                    stage, out_hbm.at[pl.ds(n_win * rows_per_win, n_tail)])

        def v0_of(c):
            return pl.multiple_of(c * 128, 128)

        def issue_read(c, slot):
            pltpu.async_copy(
                t_hbm.at[:, pl.ds(v0_of(c), 128)], win_v.at[slot], gsem)

        def wait_read(slot):
            pltpu.make_async_copy(
                t_hbm.at[:, pl.ds(0, 128)], win_v.at[slot], gsem).wait()

        def transpose_block(slot):
            src = win_v.at[slot]
            dst = trs_v.at[slot]
            for vl in range(128):
                for ge in range(d // _L):
                    f = vl * d + ge * _L
                    vals = plsc.load_gather(
                        src, [lane + ge * _L, jnp.full((_L,), vl, jnp.int32)])
                    dst[f >> 7, pl.ds(f & 127, _L)] = vals

        def issue_write(c, slot):
            row0 = pl.multiple_of(c * rows_per_win, rows_per_win)
            pltpu.async_copy(
                trs_v.at[slot], out_hbm.at[pl.ds(row0, rows_per_win)], ssem)

        def wait_write(slot):
            pltpu.make_async_copy(
                trs_v.at[slot], out_hbm.at[pl.ds(0, rows_per_win)], ssem
            ).wait()

        @pl.when(n_mine > 0)
        def _():
            issue_read(wid, 0)

            def body(k, carry):
                slot = lax.rem(k, 2)
                c = wid + k * _NW

                @pl.when(k + 1 < n_mine)
                def _():
                    issue_read(wid + (k + 1) * _NW, 1 - slot)

                wait_read(slot)

                @pl.when(k >= 2)
                def _():
                    wait_write(slot)

                transpose_block(slot)
                issue_write(c, slot)
                return carry

            lax.fori_loop(0, n_mine, body, 0)
            wait_write(0)

            @pl.when(n_mine > 1)
            def _():
                wait_write(1)

    return transpose_kernel


def _make_gather(b0, s_dim, v, d):
    mesh = plsc.VectorSubcoreMesh(core_axis_name="c", subcore_axis_name="s")
    n_ch = s_dim

    @functools.partial(
        pl.kernel,
        mesh=mesh,
        compiler_params=pltpu.CompilerParams(needs_layout_passes=False),
        out_type=jax.ShapeDtypeStruct((s_dim, d, b0), jnp.float32),
        scratch_types=[
            pltpu.VMEM((n_ch, _CH), jnp.int32),
            pltpu.VMEM((4, _CH), jnp.int32),
            pltpu.VMEM((4, _CH, 128), jnp.float32),
            pltpu.VMEM((2, d, _CH), jnp.float32),
            pltpu.SemaphoreType.DMA,
            pltpu.SemaphoreType.DMA,
            pltpu.SemaphoreType.DMA,
        ],
    )
    def gather_kernel(x_hbm, table_hbm, out_hbm, idx_v, idxq_v, rows_v,
                      out_v, isem, gsem, ssem):
        wid = lax.axis_index("s") * _NC + lax.axis_index("c")
        bbase = pl.multiple_of(wid * _CH, _CH)
        lane = lax.iota(jnp.int32, _L)

        for s in range(n_ch):
            pltpu.async_copy(
                x_hbm.at[s, pl.ds(bbase, _CH)], idx_v.at[s], isem)
        for s in range(n_ch):
            pltpu.make_async_copy(
                x_hbm.at[0, pl.ds(0, _CH)], idx_v.at[0], isem).wait()

        def issue_gather(chunk, slot):
            for g in range(_CH // _L):
                sl = pl.ds(g * _L, _L)
                idxq_v[slot, sl] = lax.shift_right_logical(idx_v[chunk, sl], 2)
            pltpu.async_copy(
                table_hbm.at[idxq_v.at[slot]], rows_v.at[slot], gsem)

        def wait_gather(slot):
            pltpu.make_async_copy(
                table_hbm.at[idxq_v.at[0]], rows_v.at[slot], gsem).wait()

        def extract(chunk, slot, oslot):
            src = rows_v.at[slot]
            dst = out_v.at[oslot]
            for g in range(_CH // _L):
                sl = pl.ds(g * _L, _L)
                qcol = lax.shift_left(
                    lax.bitwise_and(idx_v[chunk, sl], 3), 5)
                row_g = lane + g * _L
                for e in range(d):
                    vals = plsc.load_gather(src, [row_g, qcol + e])
                    dst[e, pl.ds(g * _L, _L)] = vals

        def issue_scatter(chunk, oslot):
            pltpu.async_copy(
                out_v.at[oslot],
                out_hbm.at[chunk, :, pl.ds(bbase, _CH)], ssem)

        def wait_scatter(oslot):
            pltpu.make_async_copy(
                out_v.at[oslot],
                out_hbm.at[0, :, pl.ds(bbase, _CH)], ssem).wait()

        issue_gather(0, 0)
        issue_gather(1, 1)

        def body(i, carry):
            b = lax.rem(i, 4)
            oslot = lax.rem(i, 2)

            @pl.when(i + 2 < n_ch)
            def _():
                issue_gather(i + 2, lax.rem(i + 2, 4))

            wait_gather(b)

            @pl.when(i >= 2)
            def _():
                wait_scatter(oslot)

            extract(i, b, oslot)
            issue_scatter(i, oslot)
            return carry

        lax.fori_loop(0, n_ch, body, 0)
        wait_scatter(0)
        wait_scatter(1)

    return gather_kernel


def kernel(x, table):
    b0, s_dim = x.shape
    v, d = table.shape
    tT = jnp.transpose(table)
    xT = jnp.transpose(x)
    tail = table[(v // 128) * 128:, :].reshape(-1, 128)
    scratch = _make_transpose(v, d)(tT, tail)
    out3 = _make_gather(b0, s_dim, v, d)(xT, scratch)
    return jnp.transpose(out3, (2, 0, 1))


# speedup vs baseline: 1.1631x; 1.1631x over previous
import functools

import jax
import jax.numpy as jnp
from jax import lax
from jax.experimental import pallas as pl
from jax.experimental.pallas import tpu as pltpu
from jax.experimental.pallas import tpu_sc as plsc

_NC = 2
_NS = 16
_NW = _NC * _NS
_L = 16
_CH = 128


def _make_transpose(v, d):
    mesh = plsc.VectorSubcoreMesh(core_axis_name="c", subcore_axis_name="s")
    n_win = v // 128
    n_tail = (v % 128) * d // 128
    rows_per_win = 128 * d // 128

    @functools.partial(
        pl.kernel,
        mesh=mesh,
        compiler_params=pltpu.CompilerParams(needs_layout_passes=False),
        out_type=jax.ShapeDtypeStruct((v * d,), jnp.float32),
        scratch_types=[
            pltpu.VMEM((2 * d * 128,), jnp.float32),
            pltpu.VMEM((2 * 128 * d,), jnp.float32),
            pltpu.SemaphoreType.DMA,
            pltpu.SemaphoreType.DMA,
        ],
    )
    def transpose_kernel(t_hbm, tail_hbm, out_hbm, win_v, trs_v, gsem, ssem):
        wid = lax.axis_index("s") * _NC + lax.axis_index("c")
        lane = lax.iota(jnp.int32, _L)
        lane_d = lane * d
        n_mine = (n_win - wid + _NW - 1) // _NW

        blk = d * 128

        if n_tail:
            @pl.when(wid == 0)
            def _():
                stage = trs_v.at[pl.ds(0, n_tail * 128)]
                pltpu.sync_copy(tail_hbm, stage)
                pltpu.sync_copy(
                    stage,
                    out_hbm.at[pl.ds(n_win * blk, n_tail * 128)])

        def win_base(slot):
            return pl.multiple_of(slot * blk, blk)

        def issue_read(c, slot):
            v0 = pl.multiple_of(c * 128, 128)
            base = win_base(slot)
            for e in range(d):
                pltpu.async_copy(
                    t_hbm.at[e, pl.ds(v0, 128)],
                    win_v.at[pl.ds(base + e * 128, 128)], gsem)

        def wait_read(slot):
            pltpu.make_async_copy(
                t_hbm.at[0, pl.ds(0, blk)],
                win_v.at[pl.ds(win_base(slot), blk)], gsem).wait()

        def transpose_block(slot):
            wb = slot * blk
            tb = slot * blk
            for e in range(d):
                for j in range(128 // _L):
                    vals = win_v[
                        pl.ds(pl.multiple_of(wb + e * 128 + j * _L, _L), _L)]
                    plsc.store_scatter(
                        trs_v, [lane_d + (tb + j * _L * d + e)], vals)

        def issue_write(c, slot):
            f0 = pl.multiple_of(c * blk, blk)
            pltpu.async_copy(
                trs_v.at[pl.ds(win_base(slot), blk)],
                out_hbm.at[pl.ds(f0, blk)], ssem)

        def wait_write(slot):
            pltpu.make_async_copy(
                trs_v.at[pl.ds(win_base(slot), blk)],
                out_hbm.at[pl.ds(0, blk)], ssem
            ).wait()

        @pl.when(n_mine > 0)
        def _():
            issue_read(wid, 0)

            def body(k, carry):
                slot = lax.rem(k, 2)
                c = wid + k * _NW

                @pl.when(k + 1 < n_mine)
                def _():
                    issue_read(wid + (k + 1) * _NW, 1 - slot)

                wait_read(slot)

                @pl.when(k >= 2)
                def _():
                    wait_write(slot)

                transpose_block(slot)
                issue_write(c, slot)
                return carry

            lax.fori_loop(0, n_mine, body, 0)
            wait_write(0)

            @pl.when(n_mine > 1)
            def _():
                wait_write(1)

    return transpose_kernel


def _make_gather(b0, s_dim, v, d):
    mesh = plsc.VectorSubcoreMesh(core_axis_name="c", subcore_axis_name="s")
    n_ch = s_dim

    @functools.partial(
        pl.kernel,
        mesh=mesh,
        compiler_params=pltpu.CompilerParams(needs_layout_passes=False),
        out_type=jax.ShapeDtypeStruct((s_dim, d, b0), jnp.float32),
        scratch_types=[
            pltpu.VMEM((n_ch, _CH), jnp.int32),
            pltpu.VMEM((4, _CH), jnp.int32),
            pltpu.VMEM((4, _CH, 128), jnp.float32),
            pltpu.VMEM((2, d, _CH), jnp.float32),
            pltpu.SemaphoreType.DMA,
            pltpu.SemaphoreType.DMA,
            pltpu.SemaphoreType.DMA,
        ],
    )
    def gather_kernel(x_hbm, table_hbm, out_hbm, idx_v, idxq_v, rows_v,
                      out_v, isem, gsem, ssem):
        wid = lax.axis_index("s") * _NC + lax.axis_index("c")
        bbase = pl.multiple_of(wid * _CH, _CH)
        lane = lax.iota(jnp.int32, _L)

        for s in range(n_ch):
            pltpu.async_copy(
                x_hbm.at[s, pl.ds(bbase, _CH)], idx_v.at[s], isem)
        for s in range(n_ch):
            pltpu.make_async_copy(
                x_hbm.at[0, pl.ds(0, _CH)], idx_v.at[0], isem).wait()

        def issue_gather(chunk, slot):
            for g in range(_CH // _L):
                sl = pl.ds(g * _L, _L)
                idxq_v[slot, sl] = lax.shift_right_logical(idx_v[chunk, sl], 2)
            pltpu.async_copy(
                table_hbm.at[idxq_v.at[slot]], rows_v.at[slot], gsem)

        def wait_gather(slot):
            pltpu.make_async_copy(
                table_hbm.at[idxq_v.at[0]], rows_v.at[slot], gsem).wait()

        def extract(chunk, slot, oslot):
            src = rows_v.at[slot]
            dst = out_v.at[oslot]
            for g in range(_CH // _L):
                sl = pl.ds(g * _L, _L)
                qcol = lax.shift_left(
                    lax.bitwise_and(idx_v[chunk, sl], 3), 5)
                row_g = lane + g * _L
                for e in range(d):
                    vals = plsc.load_gather(src, [row_g, qcol + e])
                    dst[e, pl.ds(g * _L, _L)] = vals

        def issue_scatter(chunk, oslot):
            pltpu.async_copy(
                out_v.at[oslot],
                out_hbm.at[chunk, :, pl.ds(bbase, _CH)], ssem)

        def wait_scatter(oslot):
            pltpu.make_async_copy(
                out_v.at[oslot],
                out_hbm.at[0, :, pl.ds(bbase, _CH)], ssem).wait()

        issue_gather(0, 0)
        issue_gather(1, 1)

        def body(i, carry):
            b = lax.rem(i, 4)
            oslot = lax.rem(i, 2)

            @pl.when(i + 2 < n_ch)
            def _():
                issue_gather(i + 2, lax.rem(i + 2, 4))

            wait_gather(b)

            @pl.when(i >= 2)
            def _():
                wait_scatter(oslot)

            extract(i, b, oslot)
            issue_scatter(i, oslot)
            return carry

        lax.fori_loop(0, n_ch, body, 0)
        wait_scatter(0)
        wait_scatter(1)

    return gather_kernel


def kernel(x, table):
    b0, s_dim = x.shape
    v, d = table.shape
    tT = jnp.transpose(table)
    xT = jnp.transpose(x)
    tail = table[(v // 128) * 128:, :].reshape(-1)
    scratch = _make_transpose(v, d)(tT, tail).reshape(v * d // 128, 128)
    out3 = _make_gather(b0, s_dim, v, d)(xT, scratch)
    return jnp.transpose(out3, (2, 0, 1))


# speedup vs baseline: 2.6785x; 2.3029x over previous
import functools

import jax
import jax.numpy as jnp
from jax import lax
from jax.experimental import pallas as pl
from jax.experimental.pallas import tpu as pltpu
from jax.experimental.pallas import tpu_sc as plsc

_NC = 2
_NS = 16
_NW = _NC * _NS
_L = 16
_CH = 128


def _make_transpose(v, d):
    mesh = plsc.VectorSubcoreMesh(core_axis_name="c", subcore_axis_name="s")
    n_win = v // 128
    n_tail = (v % 128) * d // 128
    rows_per_win = 128 * d // 128

    @functools.partial(
        pl.kernel,
        mesh=mesh,
        compiler_params=pltpu.CompilerParams(needs_layout_passes=False),
        out_type=jax.ShapeDtypeStruct((v * d,), jnp.float32),
        scratch_types=[
            pltpu.VMEM((2 * d * 128,), jnp.float32),
            pltpu.VMEM((2 * 128 * d,), jnp.float32),
            pltpu.SemaphoreType.DMA,
            pltpu.SemaphoreType.DMA,
        ],
    )
    def transpose_kernel(t_hbm, tail_hbm, out_hbm, win_v, trs_v, gsem, ssem):
        wid = lax.axis_index("s") * _NC + lax.axis_index("c")
        lane = lax.iota(jnp.int32, _L)
        lane_d = lane * d
        n_mine = (n_win - wid + _NW - 1) // _NW

        blk = d * 128

        if n_tail:
            @pl.when(wid == 0)
            def _():
                stage = trs_v.at[pl.ds(0, n_tail * 128)]
                pltpu.sync_copy(tail_hbm, stage)
                pltpu.sync_copy(
                    stage,
                    out_hbm.at[pl.ds(n_win * blk, n_tail * 128)])

        def win_base(slot):
            return pl.multiple_of(slot * blk, blk)

        def issue_read(c, slot):
            v0 = pl.multiple_of(c * 128, 128)
            base = win_base(slot)
            for e in range(d):
                pltpu.async_copy(
                    t_hbm.at[e, pl.ds(v0, 128)],
                    win_v.at[pl.ds(base + e * 128, 128)], gsem)

        def wait_read(slot):
            pltpu.make_async_copy(
                t_hbm.at[0, pl.ds(0, blk)],
                win_v.at[pl.ds(win_base(slot), blk)], gsem).wait()

        lane_pat = (lane >> 2) * 128 + (lane & 3)

        def transpose_block(slot):
            wb = slot * blk
            tb = slot * blk
            for e in range(d):
                for j in range(128 // _L):
                    vals = win_v[
                        pl.ds(pl.multiple_of(wb + e * 128 + j * _L, _L), _L)]
                    plsc.store_scatter(
                        trs_v,
                        [lane_pat + (tb + j * (_L // 4) * 128 + e * 4)],
                        vals)

        def issue_write(c, slot):
            f0 = pl.multiple_of(c * blk, blk)
            pltpu.async_copy(
                trs_v.at[pl.ds(win_base(slot), blk)],
                out_hbm.at[pl.ds(f0, blk)], ssem)

        def wait_write(slot):
            pltpu.make_async_copy(
                trs_v.at[pl.ds(win_base(slot), blk)],
                out_hbm.at[pl.ds(0, blk)], ssem
            ).wait()

        @pl.when(n_mine > 0)
        def _():
            issue_read(wid, 0)

            def body(k, carry):
                slot = lax.rem(k, 2)
                c = wid + k * _NW

                @pl.when(k + 1 < n_mine)
                def _():
                    issue_read(wid + (k + 1) * _NW, 1 - slot)

                wait_read(slot)

                @pl.when(k >= 2)
                def _():
                    wait_write(slot)

                transpose_block(slot)
                issue_write(c, slot)
                return carry

            lax.fori_loop(0, n_mine, body, 0)
            wait_write(0)

            @pl.when(n_mine > 1)
            def _():
                wait_write(1)

    return transpose_kernel


def _make_gather(b0, s_dim, v, d):
    mesh = plsc.VectorSubcoreMesh(core_axis_name="c", subcore_axis_name="s")
    n_ch = s_dim

    @functools.partial(
        pl.kernel,
        mesh=mesh,
        compiler_params=pltpu.CompilerParams(needs_layout_passes=False),
        out_type=jax.ShapeDtypeStruct((s_dim, d, b0), jnp.float32),
        scratch_types=[
            pltpu.VMEM((n_ch, _CH), jnp.int32),
            pltpu.VMEM((4, _CH), jnp.int32),
            pltpu.VMEM((4, _CH, 128), jnp.float32),
            pltpu.VMEM((2, d, _CH), jnp.float32),
            pltpu.SemaphoreType.DMA,
            pltpu.SemaphoreType.DMA,
            pltpu.SemaphoreType.DMA,
        ],
    )
    def gather_kernel(x_hbm, table_hbm, out_hbm, idx_v, idxq_v, rows_v,
                      out_v, isem, gsem, ssem):
        wid = lax.axis_index("s") * _NC + lax.axis_index("c")
        bbase = pl.multiple_of(wid * _CH, _CH)
        lane = lax.iota(jnp.int32, _L)

        for s in range(n_ch):
            pltpu.async_copy(
                x_hbm.at[s, pl.ds(bbase, _CH)], idx_v.at[s], isem)
        for s in range(n_ch):
            pltpu.make_async_copy(
                x_hbm.at[0, pl.ds(0, _CH)], idx_v.at[0], isem).wait()

        def issue_gather(chunk, slot):
            for g in range(_CH // _L):
                sl = pl.ds(g * _L, _L)
                idxq_v[slot, sl] = lax.shift_right_logical(idx_v[chunk, sl], 2)
            pltpu.async_copy(
                table_hbm.at[idxq_v.at[slot]], rows_v.at[slot], gsem)

        def wait_gather(slot):
            pltpu.make_async_copy(
                table_hbm.at[idxq_v.at[0]], rows_v.at[slot], gsem).wait()

        def extract(chunk, slot, oslot):
            src = rows_v.at[slot]
            dst = out_v.at[oslot]
            for g in range(_CH // _L):
                sl = pl.ds(g * _L, _L)
                qcol = lax.bitwise_and(idx_v[chunk, sl], 3)
                row_g = lane + g * _L
                for e in range(d):
                    vals = plsc.load_gather(src, [row_g, qcol + e * 4])
                    dst[e, pl.ds(g * _L, _L)] = vals

        def issue_scatter(chunk, oslot):
            pltpu.async_copy(
                out_v.at[oslot],
                out_hbm.at[chunk, :, pl.ds(bbase, _CH)], ssem)

        def wait_scatter(oslot):
            pltpu.make_async_copy(
                out_v.at[oslot],
                out_hbm.at[0, :, pl.ds(bbase, _CH)], ssem).wait()

        issue_gather(0, 0)
        issue_gather(1, 1)

        def body(i, carry):
            b = lax.rem(i, 4)
            oslot = lax.rem(i, 2)

            @pl.when(i + 2 < n_ch)
            def _():
                issue_gather(i + 2, lax.rem(i + 2, 4))

            wait_gather(b)

            @pl.when(i >= 2)
            def _():
                wait_scatter(oslot)

            extract(i, b, oslot)
            issue_scatter(i, oslot)
            return carry

        lax.fori_loop(0, n_ch, body, 0)
        wait_scatter(0)
        wait_scatter(1)

    return gather_kernel


def kernel(x, table):
    b0, s_dim = x.shape
    v, d = table.shape
    tT = jnp.transpose(table)
    xT = jnp.transpose(x)
    tail = (table[(v // 128) * 128:, :]
            .reshape(-1, 4, d).transpose(0, 2, 1).reshape(-1))
    scratch = _make_transpose(v, d)(tT, tail).reshape(v * d // 128, 128)
    out3 = _make_gather(b0, s_dim, v, d)(xT, scratch)
    return jnp.transpose(out3, (2, 0, 1))


# speedup vs baseline: 2.6833x; 1.0018x over previous
import functools

import jax
import jax.numpy as jnp
from jax import lax
from jax.experimental import pallas as pl
from jax.experimental.pallas import tpu as pltpu
from jax.experimental.pallas import tpu_sc as plsc

_NC = 2
_NS = 16
_NW = _NC * _NS
_L = 16
_CH = 128


def _make_transpose(v, d):
    mesh = plsc.VectorSubcoreMesh(core_axis_name="c", subcore_axis_name="s")
    n_win = v // 128
    n_tail = (v % 128) * d // 128
    rows_per_win = 128 * d // 128

    @functools.partial(
        pl.kernel,
        mesh=mesh,
        compiler_params=pltpu.CompilerParams(needs_layout_passes=False),
        out_type=jax.ShapeDtypeStruct((v * d,), jnp.float32),
        scratch_types=[
            pltpu.VMEM((2 * d * 128,), jnp.float32),
            pltpu.VMEM((2 * 128 * d,), jnp.float32),
            pltpu.SemaphoreType.DMA,
            pltpu.SemaphoreType.DMA,
        ],
    )
    def transpose_kernel(t_hbm, tail_hbm, out_hbm, win_v, trs_v, gsem, ssem):
        wid = lax.axis_index("s") * _NC + lax.axis_index("c")
        lane = lax.iota(jnp.int32, _L)
        lane_d = lane * d
        n_mine = (n_win - wid + _NW - 1) // _NW

        blk = d * 128

        if n_tail:
            @pl.when(wid == 0)
            def _():
                stage = trs_v.at[pl.ds(0, n_tail * 128)]
                pltpu.sync_copy(tail_hbm, stage)
                pltpu.sync_copy(
                    stage,
                    out_hbm.at[pl.ds(n_win * blk, n_tail * 128)])

        def win_base(slot):
            return pl.multiple_of(slot * blk, blk)

        def issue_read(c, slot):
            v0 = pl.multiple_of(c * 128, 128)
            base = win_base(slot)
            for e in range(d):
                pltpu.async_copy(
                    t_hbm.at[e, pl.ds(v0, 128)],
                    win_v.at[pl.ds(base + e * 128, 128)], gsem)

        def wait_read(slot):
            pltpu.make_async_copy(
                t_hbm.at[0, pl.ds(0, blk)],
                win_v.at[pl.ds(win_base(slot), blk)], gsem).wait()

        lane_pat = (lane >> 2) * 128 + (lane & 3)

        def transpose_block(slot):
            wb = slot * blk
            tb = slot * blk
            for e in range(d):
                for j in range(128 // _L):
                    vals = win_v[
                        pl.ds(pl.multiple_of(wb + e * 128 + j * _L, _L), _L)]
                    plsc.store_scatter(
                        trs_v,
                        [lane_pat + (tb + j * (_L // 4) * 128 + e * 4)],
                        vals)

        def issue_write(c, slot):
            f0 = pl.multiple_of(c * blk, blk)
            pltpu.async_copy(
                trs_v.at[pl.ds(win_base(slot), blk)],
                out_hbm.at[pl.ds(f0, blk)], ssem)

        def wait_write(slot):
            pltpu.make_async_copy(
                trs_v.at[pl.ds(win_base(slot), blk)],
                out_hbm.at[pl.ds(0, blk)], ssem
            ).wait()

        @pl.when(n_mine > 0)
        def _():
            issue_read(wid, 0)

            def body(k, carry):
                slot = lax.rem(k, 2)
                c = wid + k * _NW

                @pl.when(k + 1 < n_mine)
                def _():
                    issue_read(wid + (k + 1) * _NW, 1 - slot)

                wait_read(slot)

                @pl.when(k >= 2)
                def _():
                    wait_write(slot)

                transpose_block(slot)
                issue_write(c, slot)
                return carry

            lax.fori_loop(0, n_mine, body, 0)
            wait_write(0)

            @pl.when(n_mine > 1)
            def _():
                wait_write(1)

    return transpose_kernel


def _make_gather(b0, s_dim, v, d):
    mesh = plsc.VectorSubcoreMesh(core_axis_name="c", subcore_axis_name="s")
    n_ch = s_dim

    @functools.partial(
        pl.kernel,
        mesh=mesh,
        compiler_params=pltpu.CompilerParams(needs_layout_passes=False),
        out_type=jax.ShapeDtypeStruct((s_dim, d, b0), jnp.float32),
        scratch_types=[
            pltpu.VMEM((n_ch, _CH), jnp.int32),
            pltpu.VMEM((6, _CH), jnp.int32),
            pltpu.VMEM((6, _CH, 128), jnp.float32),
            pltpu.VMEM((2, d, _CH), jnp.float32),
            pltpu.SemaphoreType.DMA,
            pltpu.SemaphoreType.DMA,
            pltpu.SemaphoreType.DMA,
        ],
    )
    def gather_kernel(x_hbm, table_hbm, out_hbm, idx_v, idxq_v, rows_v,
                      out_v, isem, gsem, ssem):
        wid = lax.axis_index("s") * _NC + lax.axis_index("c")
        bbase = pl.multiple_of(wid * _CH, _CH)
        lane = lax.iota(jnp.int32, _L)

        for s in range(n_ch):
            pltpu.async_copy(
                x_hbm.at[s, pl.ds(bbase, _CH)], idx_v.at[s], isem)
        for s in range(n_ch):
            pltpu.make_async_copy(
                x_hbm.at[0, pl.ds(0, _CH)], idx_v.at[0], isem).wait()

        def issue_gather(chunk, slot):
            for g in range(_CH // _L):
                sl = pl.ds(g * _L, _L)
                idxq_v[slot, sl] = lax.shift_right_logical(idx_v[chunk, sl], 2)
            pltpu.async_copy(
                table_hbm.at[idxq_v.at[slot]], rows_v.at[slot], gsem)

        def wait_gather(slot):
            pltpu.make_async_copy(
                table_hbm.at[idxq_v.at[0]], rows_v.at[slot], gsem).wait()

        def extract(chunk, slot, oslot):
            src = rows_v.at[slot]
            dst = out_v.at[oslot]
            for g in range(_CH // _L):
                sl = pl.ds(g * _L, _L)
                qcol = lax.bitwise_and(idx_v[chunk, sl], 3)
                row_g = lane + g * _L
                for e in range(d):
                    vals = plsc.load_gather(src, [row_g, qcol + e * 4])
                    dst[e, pl.ds(g * _L, _L)] = vals

        def issue_scatter(chunk, oslot):
            pltpu.async_copy(
                out_v.at[oslot],
                out_hbm.at[chunk, :, pl.ds(bbase, _CH)], ssem)

        def wait_scatter(oslot):
            pltpu.make_async_copy(
                out_v.at[oslot],
                out_hbm.at[0, :, pl.ds(bbase, _CH)], ssem).wait()

        issue_gather(0, 0)
        issue_gather(1, 1)
        issue_gather(2, 2)

        def body(i, carry):
            b = lax.rem(i, 6)
            oslot = lax.rem(i, 2)

            @pl.when(i + 3 < n_ch)
            def _():
                issue_gather(i + 3, lax.rem(i + 3, 6))

            wait_gather(b)

            @pl.when(i >= 2)
            def _():
                wait_scatter(oslot)

            extract(i, b, oslot)
            issue_scatter(i, oslot)
            return carry

        lax.fori_loop(0, n_ch, body, 0)
        wait_scatter(0)
        wait_scatter(1)

    return gather_kernel


def kernel(x, table):
    b0, s_dim = x.shape
    v, d = table.shape
    tT = jnp.transpose(table)
    xT = jnp.transpose(x)
    tail = (table[(v // 128) * 128:, :]
            .reshape(-1, 4, d).transpose(0, 2, 1).reshape(-1))
    scratch = _make_transpose(v, d)(tT, tail).reshape(v * d // 128, 128)
    out3 = _make_gather(b0, s_dim, v, d)(xT, scratch)
    return jnp.transpose(out3, (2, 0, 1))
